# baseline (speedup 1.0000x reference)
import functools

import jax
import jax.numpy as jnp
from jax.experimental import pallas as pl
from jax.experimental.pallas import tpu as pltpu

_D = 64
_T = 10
_N = 100000
_NBLK = 6272
_K = 16
_NPAD = _K * _NBLK


def _gam_kernel(h_ref, cls_ref, rank_ref, pred_ref, attn_ref, logits_ref):
    i = pl.program_id(0)
    h = h_ref[:, :]
    logits_ref[:, pl.ds(i * _NBLK, _NBLK)] = jnp.dot(
        h, rank_ref[:, :], preferred_element_type=jnp.float32
    )

    @pl.when(i == _K - 1)
    def _finalize():
        pred_ref[:, :] = jnp.dot(h, cls_ref[:, :],
                                 preferred_element_type=jnp.float32)
        cols = jax.lax.broadcasted_iota(jnp.int32, (1, _NPAD), 1)
        logits = jnp.where(cols < _N, logits_ref[:, :], -jnp.inf)
        m = jnp.max(logits)
        e = jnp.exp(logits - m)
        s = jnp.sum(e)
        attn_ref[:, :] = (e / s)[:, :_N]


@jax.jit
def kernel(hidden_state, theta_classification, theta_rank):
    h = hidden_state.reshape(1, _D)
    pred, attn = pl.pallas_call(
        _gam_kernel,
        grid=(_K,),
        in_specs=[
            pl.BlockSpec((1, _D), lambda i: (0, 0)),
            pl.BlockSpec((_D, _T), lambda i: (0, 0)),
            pl.BlockSpec((_D, _NBLK), lambda i: (0, i)),
        ],
        out_specs=[
            pl.BlockSpec((1, _T), lambda i: (0, 0)),
            pl.BlockSpec((1, _N), lambda i: (0, 0)),
        ],
        out_shape=[
            jax.ShapeDtypeStruct((1, _T), jnp.float32),
            jax.ShapeDtypeStruct((1, _N), jnp.float32),
        ],
        scratch_shapes=[pltpu.VMEM((1, _NPAD), jnp.float32)],
    )(h, theta_classification, theta_rank)
    return (pred, attn)
